# baseline (speedup 1.0000x reference)
import functools

import jax
import jax.numpy as jnp
from jax import lax
from jax.experimental import pallas as pl
from jax.experimental.pallas import tpu as pltpu
from jax.experimental.pallas import tpu_sc as plsc

_BATCH = 4
_ROWS = 8192
_DIM = 1024
_NC = 2
_NS = 16
_NW = _NC * _NS
_RPW = _ROWS // _NW


def _build():
    mesh = plsc.VectorSubcoreMesh(core_axis_name="c", subcore_axis_name="s")

    @functools.partial(
        pl.kernel,
        mesh=mesh,
        out_type=jax.ShapeDtypeStruct((_BATCH, _ROWS, _DIM), jnp.float32),
        scratch_types=[pltpu.SemaphoreType.DMA],
    )
    def bcast(table_hbm, out_hbm, sem):
        wid = lax.axis_index("s") * _NC + lax.axis_index("c")
        base = wid * _RPW
        copies = [
            pltpu.make_async_copy(
                table_hbm.at[pl.ds(base, _RPW)],
                out_hbm.at[b, pl.ds(base, _RPW)],
                sem,
            )
            for b in range(_BATCH)
        ]
        for c in copies:
            c.start()
        for c in copies:
            c.wait()

    return bcast


_BCAST = _build()


def kernel(batch, embed_weight):
    del batch
    return _BCAST(embed_weight)


# speedup vs baseline: 54.6523x; 54.6523x over previous
import functools

import jax
import jax.numpy as jnp
from jax import lax
from jax.experimental import pallas as pl
from jax.experimental.pallas import tpu as pltpu
from jax.experimental.pallas import tpu_sc as plsc

_BATCH = 4
_ROWS = 8192
_DIM = 1024
_NC = 2
_NS = 16
_NW = _NC * _NS
_RPW = _ROWS // _NW


_CH = 32
_NCHUNK = _RPW // _CH


def _build():
    mesh = plsc.VectorSubcoreMesh(core_axis_name="c", subcore_axis_name="s")

    @functools.partial(
        pl.kernel,
        mesh=mesh,
        out_type=jax.ShapeDtypeStruct((_BATCH, _ROWS, _DIM), jnp.float32),
        scratch_types=[
            pltpu.VMEM((_CH, _DIM), jnp.float32),
            pltpu.VMEM((_CH, _DIM), jnp.float32),
            pltpu.SemaphoreType.DMA,
            pltpu.SemaphoreType.DMA,
        ],
    )
    def bcast(table_hbm, out_hbm, buf0, buf1, lsem, ssem):
        wid = lax.axis_index("s") * _NC + lax.axis_index("c")
        base = wid * _RPW
        bufs = (buf0, buf1)

        def load(i):
            return pltpu.make_async_copy(
                table_hbm.at[pl.ds(base + i * _CH, _CH)], bufs[i % 2], lsem)

        def store(i, b):
            return pltpu.make_async_copy(
                bufs[i % 2], out_hbm.at[b, pl.ds(base + i * _CH, _CH)], ssem)

        loads = [load(i) for i in range(_NCHUNK)]
        stores = [[store(i, b) for b in range(_BATCH)] for i in range(_NCHUNK)]

        loads[0].start()
        for i in range(_NCHUNK):
            if i + 1 < _NCHUNK:
                if i >= 1:
                    for c in stores[i - 1]:
                        c.wait()
                loads[i + 1].start()
            loads[i].wait()
            for c in stores[i]:
                c.start()
        for c in stores[_NCHUNK - 2]:
            c.wait()
        for c in stores[_NCHUNK - 1]:
            c.wait()

    return bcast


_BCAST = _build()


def kernel(batch, embed_weight):
    del batch
    return _BCAST(embed_weight)
